# Initial kernel scaffold; baseline (speedup 1.0000x reference)
import jax
import jax.numpy as jnp
from jax.experimental import pallas as pl


def kernel(x, expert_mus, expert_log_sigmas, W1, b1, W2, b2):
    raise NotImplementedError("write your pallas kernel here")


# speedup vs baseline: 3.5156x; 3.5156x over previous
import functools

import jax
import jax.numpy as jnp
import numpy as np
from jax import lax
from jax.experimental import pallas as pl
from jax.experimental.pallas import tpu as pltpu
from jax.experimental.pallas import tpu_sc as plsc

E = 8
TOP_K = 2
D_IN = 1024
D_HID = 2048
D_OUT = 1024
T = 2048

TM = 128
NPAD = 5120
NT = NPAD // TM
NW = 32
TOK_PER_W = T // NW
CH = 16
NCH = TOK_PER_W // CH

_LOG2PI = 1.8378770664093453


def _router_body(x_ref, mu_ref, ls_ref, lp_ref, w_ref, idx_ref, pos_ref, cnt_ref):
    x = x_ref[...]
    cols = []
    for e in range(E):
        mu = mu_ref[e:e + 1, :]
        ls = ls_ref[e:e + 1, :]
        inv_sigma = jnp.exp(-ls)
        z = (x - mu) * inv_sigma
        s = jnp.sum(z * z, axis=1, keepdims=True)
        sls = jnp.sum(ls, axis=1, keepdims=True)
        cols.append(-0.5 * s - sls - (0.5 * _LOG2PI * D_IN))
    lp = jnp.concatenate(cols, axis=1)
    lp_ref[...] = lp

    iota_e = lax.broadcasted_iota(jnp.int32, (T, E), 1)
    m1 = jnp.max(lp, axis=1, keepdims=True)
    i1 = jnp.min(jnp.where(lp == m1, iota_e, E), axis=1, keepdims=True)
    masked = jnp.where(iota_e == i1, -jnp.inf, lp)
    m2 = jnp.max(masked, axis=1, keepdims=True)
    i2 = jnp.min(jnp.where(masked == m2, iota_e, E), axis=1, keepdims=True)

    p2 = jnp.exp(m2 - m1)
    inv = 1.0 / (1.0 + p2)
    w_ref[...] = jnp.concatenate([inv, p2 * inv], axis=1)
    idx_ref[...] = jnp.concatenate([i1, i2], axis=1)

    c1 = (iota_e == i1).astype(jnp.int32)
    c2 = (iota_e == i2).astype(jnp.int32)
    a = c1 + c2
    s_inc = a
    k = 1
    while k < T:
        shifted = jnp.concatenate(
            [jnp.zeros((k, E), jnp.int32), s_inc[: T - k, :]], axis=0)
        s_inc = s_inc + shifted
        k *= 2
    s_exc = s_inc - a
    counts = s_inc[T - 1: T, :]
    cnt_ref[...] = counts

    padded = ((counts + (TM - 1)) // TM) * TM
    p_exc = jnp.concatenate(
        [jnp.zeros((1, 1), jnp.int32), padded[:, : E - 1]], axis=1)
    k = 1
    while k < E:
        shifted = jnp.concatenate(
            [jnp.zeros((1, k), jnp.int32), p_exc[:, : E - k]], axis=1)
        p_exc = p_exc + shifted
        k *= 2
    slot = s_exc + p_exc
    pos1 = jnp.sum(c1 * slot, axis=1, keepdims=True)
    pos2 = jnp.sum(c2 * slot, axis=1, keepdims=True)
    pos_ref[...] = jnp.concatenate([pos1, pos2], axis=1)


def _router(xf, mus, lsig):
    return pl.pallas_call(
        _router_body,
        out_shape=(
            jax.ShapeDtypeStruct((T, E), jnp.float32),
            jax.ShapeDtypeStruct((T, TOP_K), jnp.float32),
            jax.ShapeDtypeStruct((T, TOP_K), jnp.int32),
            jax.ShapeDtypeStruct((T, TOP_K), jnp.int32),
            jax.ShapeDtypeStruct((1, E), jnp.int32),
        ),
    )(xf, mus, lsig)


_SC_MESH = plsc.VectorSubcoreMesh(core_axis_name="c", subcore_axis_name="s")


@functools.partial(
    pl.kernel,
    mesh=_SC_MESH,
    out_type=jax.ShapeDtypeStruct((NPAD, D_IN), jnp.float32),
    scratch_types=[
        pltpu.VMEM((NCH, CH), jnp.int32),
        pltpu.VMEM((NCH, CH), jnp.int32),
        pltpu.VMEM((CH, D_IN), jnp.float32),
        pltpu.SemaphoreType.DMA,
        pltpu.SemaphoreType.DMA,
    ],
)
def _sc_scatter(x_hbm, p0_hbm, p1_hbm, out_hbm, i0_v, i1_v, xbuf, sem0, sem1):
    wid = lax.axis_index("s") * 2 + lax.axis_index("c")
    pltpu.sync_copy(p0_hbm.at[wid], i0_v)
    pltpu.sync_copy(p1_hbm.at[wid], i1_v)
    for ch in range(NCH):
        base = wid * TOK_PER_W + ch * CH
        pltpu.sync_copy(x_hbm.at[pl.ds(base, CH)], xbuf)
        cp0 = pltpu.async_copy(xbuf, out_hbm.at[i0_v.at[ch]], sem0)
        cp1 = pltpu.async_copy(xbuf, out_hbm.at[i1_v.at[ch]], sem1)
        cp0.wait()
        cp1.wait()


@functools.partial(
    pl.kernel,
    mesh=_SC_MESH,
    out_type=(
        jax.ShapeDtypeStruct((T, D_OUT), jnp.float32),
        jax.ShapeDtypeStruct((T, D_OUT), jnp.float32),
    ),
    scratch_types=[
        pltpu.VMEM((NCH, CH), jnp.int32),
        pltpu.VMEM((NCH, CH), jnp.int32),
        pltpu.VMEM((CH, D_OUT), jnp.float32),
        pltpu.VMEM((CH, D_OUT), jnp.float32),
        pltpu.SemaphoreType.DMA,
        pltpu.SemaphoreType.DMA,
    ],
)
def _sc_gather(o_hbm, p0_hbm, p1_hbm, g0_hbm, g1_hbm, i0_v, i1_v, b0, b1, sem0, sem1):
    wid = lax.axis_index("s") * 2 + lax.axis_index("c")
    pltpu.sync_copy(p0_hbm.at[wid], i0_v)
    pltpu.sync_copy(p1_hbm.at[wid], i1_v)
    for ch in range(NCH):
        base = wid * TOK_PER_W + ch * CH
        cp0 = pltpu.async_copy(o_hbm.at[i0_v.at[ch]], b0, sem0)
        cp1 = pltpu.async_copy(o_hbm.at[i1_v.at[ch]], b1, sem1)
        cp0.wait()
        cp1.wait()
        pltpu.sync_copy(b0, g0_hbm.at[pl.ds(base, CH)])
        pltpu.sync_copy(b1, g1_hbm.at[pl.ds(base, CH)])


_SQRT1_2 = float(1.0 / np.sqrt(2.0))


def _gelu_exact(x):
    return 0.5 * x * (1.0 + lax.erf(x * _SQRT1_2))


def _mlp_body(eids_ref, rb_ref, vld_ref, x_ref, w1_ref, b1_ref, w2_ref, b2_ref,
              o_ref, w1bf, w2bf, preve):
    i = pl.program_id(0)
    e = eids_ref[i]

    @pl.when((i == 0) | (e != preve[0]))
    def _cast():
        w1bf[...] = w1_ref[0].astype(jnp.bfloat16)
        w2bf[...] = w2_ref[0].astype(jnp.bfloat16)
        preve[0] = e

    @pl.when(vld_ref[i] == 1)
    def _compute():
        xb = x_ref[...].astype(jnp.bfloat16)
        h = jnp.dot(xb, w1bf[...], preferred_element_type=jnp.float32)
        h = _gelu_exact(h + b1_ref[0])
        o = jnp.dot(h.astype(jnp.bfloat16), w2bf[...],
                    preferred_element_type=jnp.float32)
        o_ref[...] = o + b2_ref[0]


def _grouped_mlp(x_sorted, W1, b1r, W2, b2r, eids, rb, vld):
    grid_spec = pltpu.PrefetchScalarGridSpec(
        num_scalar_prefetch=3,
        grid=(NT,),
        in_specs=[
            pl.BlockSpec((TM, D_IN), lambda i, eids, rb, vld: (rb[i], 0)),
            pl.BlockSpec((1, D_IN, D_HID), lambda i, eids, rb, vld: (eids[i], 0, 0)),
            pl.BlockSpec((1, 1, D_HID), lambda i, eids, rb, vld: (eids[i], 0, 0)),
            pl.BlockSpec((1, D_HID, D_OUT), lambda i, eids, rb, vld: (eids[i], 0, 0)),
            pl.BlockSpec((1, 1, D_OUT), lambda i, eids, rb, vld: (eids[i], 0, 0)),
        ],
        out_specs=pl.BlockSpec((TM, D_OUT), lambda i, eids, rb, vld: (rb[i], 0)),
        scratch_shapes=[
            pltpu.VMEM((D_IN, D_HID), jnp.bfloat16),
            pltpu.VMEM((D_HID, D_OUT), jnp.bfloat16),
            pltpu.SMEM((1,), jnp.int32),
        ],
    )
    return pl.pallas_call(
        _mlp_body,
        grid_spec=grid_spec,
        out_shape=jax.ShapeDtypeStruct((NPAD, D_OUT), jnp.float32),
    )(eids, rb, vld, x_sorted, W1, b1r, W2, b2r)


def _combine_body(g0_ref, g1_ref, w_ref, y_ref):
    w = w_ref[...]
    y_ref[...] = g0_ref[...] * w[:, 0:1] + g1_ref[...] * w[:, 1:2]


def _combine(g0, g1, w2d):
    grid = T // TM
    return pl.pallas_call(
        _combine_body,
        grid=(grid,),
        in_specs=[
            pl.BlockSpec((TM, D_OUT), lambda i: (i, 0)),
            pl.BlockSpec((TM, D_OUT), lambda i: (i, 0)),
            pl.BlockSpec((TM, TOP_K), lambda i: (i, 0)),
        ],
        out_specs=pl.BlockSpec((TM, D_OUT), lambda i: (i, 0)),
        out_shape=jax.ShapeDtypeStruct((T, D_OUT), jnp.float32),
    )(g0, g1, w2d)


def kernel(x, expert_mus, expert_log_sigmas, W1, b1, W2, b2):
    Bm, Tm, Din = x.shape
    xf = x.reshape(Tm, Din)

    lp, w2d, idx2d, pos2d, counts = _router(xf, expert_mus, expert_log_sigmas)

    counts1 = counts[0]
    padded = ((counts1 + (TM - 1)) // TM) * TM
    poff = jnp.concatenate([jnp.zeros((1,), jnp.int32), jnp.cumsum(padded)])
    total = poff[E]
    rows = jnp.arange(NT, dtype=jnp.int32) * TM
    eidv = jnp.sum((rows[:, None] >= poff[None, 1:]).astype(jnp.int32), axis=1)
    valid = (rows < total).astype(jnp.int32)
    last = total // TM - 1
    e_last = jnp.take(eidv, last)
    eids = jnp.where(valid == 1, eidv, e_last).astype(jnp.int32)
    rb = jnp.where(valid == 1, jnp.arange(NT, dtype=jnp.int32), last).astype(jnp.int32)

    p0 = pos2d[:, 0].reshape(NW, NCH, CH)
    p1 = pos2d[:, 1].reshape(NW, NCH, CH)

    x_sorted = _sc_scatter(xf, p0, p1)
    o_sorted = _grouped_mlp(x_sorted, W1, b1.reshape(E, 1, D_HID), W2,
                            b2.reshape(E, 1, D_OUT), eids, rb, valid)
    g0, g1 = _sc_gather(o_sorted, p0, p1)
    y = _combine(g0, g1, w2d)

    return (y.reshape(Bm, Tm, D_OUT),
            lp.reshape(Bm, Tm, E),
            w2d.reshape(Bm, Tm, TOP_K),
            idx2d.reshape(Bm, Tm, TOP_K))


# speedup vs baseline: 3.5448x; 1.0083x over previous
import functools

import jax
import jax.numpy as jnp
import numpy as np
from jax import lax
from jax.experimental import pallas as pl
from jax.experimental.pallas import tpu as pltpu
from jax.experimental.pallas import tpu_sc as plsc

E = 8
TOP_K = 2
D_IN = 1024
D_HID = 2048
D_OUT = 1024
T = 2048

TM = 128
NPAD = 5120
NT = NPAD // TM
NW = 32
TOK_PER_W = T // NW
CH = 16
NCH = TOK_PER_W // CH

_LOG2PI = 1.8378770664093453


def _router_body(x_ref, mu_ref, ls_ref, lp_ref, w_ref, idx_ref, pos_ref,
                 eids_ref, rb_ref, vld_ref):
    x = x_ref[...]
    cols = []
    for e in range(E):
        mu = mu_ref[e:e + 1, :]
        ls = ls_ref[e:e + 1, :]
        inv_sigma = jnp.exp(-ls)
        z = (x - mu) * inv_sigma
        s = jnp.sum(z * z, axis=1, keepdims=True)
        sls = jnp.sum(ls, axis=1, keepdims=True)
        cols.append(-0.5 * s - sls - (0.5 * _LOG2PI * D_IN))
    lp = jnp.concatenate(cols, axis=1)
    lp_ref[...] = lp

    iota_e = lax.broadcasted_iota(jnp.int32, (T, E), 1)
    m1 = jnp.max(lp, axis=1, keepdims=True)
    i1 = jnp.min(jnp.where(lp == m1, iota_e, E), axis=1, keepdims=True)
    masked = jnp.where(iota_e == i1, -jnp.inf, lp)
    m2 = jnp.max(masked, axis=1, keepdims=True)
    i2 = jnp.min(jnp.where(masked == m2, iota_e, E), axis=1, keepdims=True)

    p2 = jnp.exp(m2 - m1)
    inv = 1.0 / (1.0 + p2)
    w_ref[...] = jnp.concatenate([inv, p2 * inv], axis=1)
    idx_ref[...] = jnp.concatenate([i1, i2], axis=1)

    c1 = (iota_e == i1).astype(jnp.int32)
    c2 = (iota_e == i2).astype(jnp.int32)
    a = c1 + c2
    s_inc = a
    k = 1
    while k < T:
        shifted = jnp.concatenate(
            [jnp.zeros((k, E), jnp.int32), s_inc[: T - k, :]], axis=0)
        s_inc = s_inc + shifted
        k *= 2
    s_exc = s_inc - a
    counts = s_inc[T - 1: T, :]

    padded = ((counts + (TM - 1)) // TM) * TM
    p_exc = jnp.concatenate(
        [jnp.zeros((1, 1), jnp.int32), padded[:, : E - 1]], axis=1)
    k = 1
    while k < E:
        shifted = jnp.concatenate(
            [jnp.zeros((1, k), jnp.int32), p_exc[:, : E - k]], axis=1)
        p_exc = p_exc + shifted
        k *= 2
    slot = s_exc + p_exc
    pos1 = jnp.sum(c1 * slot, axis=1, keepdims=True)
    pos2 = jnp.sum(c2 * slot, axis=1, keepdims=True)
    pos_ref[...] = jnp.concatenate([pos1, pos2], axis=1)

    ends = p_exc + padded
    rows = lax.broadcasted_iota(jnp.int32, (1, NT), 1) * TM
    eidv = jnp.zeros((1, NT), jnp.int32)
    e_last = jnp.zeros((1, 1), jnp.int32)
    total = ends[:, E - 1: E]
    for e in range(E):
        eidv = eidv + (rows >= ends[:, e: e + 1]).astype(jnp.int32)
        e_last = e_last + (total - TM >= ends[:, e: e + 1]).astype(jnp.int32)
    valid = (rows < total).astype(jnp.int32)
    iota_t = lax.broadcasted_iota(jnp.int32, (1, NT), 1)
    rb_ref[...] = jnp.where(valid == 1, iota_t, total // TM - 1)
    eids_ref[...] = jnp.where(valid == 1, eidv, e_last)
    vld_ref[...] = valid


def _router(xf, mus, lsig):
    return pl.pallas_call(
        _router_body,
        out_shape=(
            jax.ShapeDtypeStruct((T, E), jnp.float32),
            jax.ShapeDtypeStruct((T, TOP_K), jnp.float32),
            jax.ShapeDtypeStruct((T, TOP_K), jnp.int32),
            jax.ShapeDtypeStruct((T, TOP_K), jnp.int32),
            jax.ShapeDtypeStruct((1, NT), jnp.int32),
            jax.ShapeDtypeStruct((1, NT), jnp.int32),
            jax.ShapeDtypeStruct((1, NT), jnp.int32),
        ),
    )(xf, mus, lsig)


_SC_MESH = plsc.VectorSubcoreMesh(core_axis_name="c", subcore_axis_name="s")


@functools.partial(
    pl.kernel,
    mesh=_SC_MESH,
    out_type=jax.ShapeDtypeStruct((NPAD, D_IN), jnp.float32),
    scratch_types=[
        pltpu.VMEM((NCH, CH), jnp.int32),
        pltpu.VMEM((NCH, CH), jnp.int32),
        pltpu.VMEM((CH, D_IN), jnp.float32),
        pltpu.VMEM((CH, D_IN), jnp.float32),
        pltpu.SemaphoreType.DMA,
        pltpu.SemaphoreType.DMA,
        pltpu.SemaphoreType.DMA,
        pltpu.SemaphoreType.DMA,
    ],
)
def _sc_scatter(x_hbm, p0_hbm, p1_hbm, out_hbm, i0_v, i1_v, xb0, xb1,
                sl0, sl1, ss0, ss1):
    wid = lax.axis_index("s") * 2 + lax.axis_index("c")
    pltpu.sync_copy(p0_hbm.at[wid], i0_v)
    pltpu.sync_copy(p1_hbm.at[wid], i1_v)
    bufs = (xb0, xb1)
    lsems = (sl0, sl1)
    ssems = (ss0, ss1)
    loads = [None] * NCH
    scats = [None] * NCH
    base0 = wid * TOK_PER_W
    loads[0] = pltpu.async_copy(x_hbm.at[pl.ds(base0, CH)], xb0, sl0)
    for ch in range(NCH):
        b = bufs[ch % 2]
        loads[ch].wait()
        if ch + 1 < NCH:
            if ch >= 1:
                scats[ch - 1][0].wait()
                scats[ch - 1][1].wait()
            loads[ch + 1] = pltpu.async_copy(
                x_hbm.at[pl.ds(base0 + (ch + 1) * CH, CH)],
                bufs[(ch + 1) % 2], lsems[(ch + 1) % 2])
        scats[ch] = (
            pltpu.async_copy(b, out_hbm.at[i0_v.at[ch]], ssems[ch % 2]),
            pltpu.async_copy(b, out_hbm.at[i1_v.at[ch]], ssems[ch % 2]),
        )
    for ch in (NCH - 2, NCH - 1):
        scats[ch][0].wait()
        scats[ch][1].wait()


@functools.partial(
    pl.kernel,
    mesh=_SC_MESH,
    out_type=(
        jax.ShapeDtypeStruct((T, D_OUT), jnp.float32),
        jax.ShapeDtypeStruct((T, D_OUT), jnp.float32),
    ),
    scratch_types=[
        pltpu.VMEM((NCH, CH), jnp.int32),
        pltpu.VMEM((NCH, CH), jnp.int32),
        pltpu.VMEM((CH, D_OUT), jnp.float32),
        pltpu.VMEM((CH, D_OUT), jnp.float32),
        pltpu.VMEM((CH, D_OUT), jnp.float32),
        pltpu.VMEM((CH, D_OUT), jnp.float32),
        pltpu.SemaphoreType.DMA,
        pltpu.SemaphoreType.DMA,
        pltpu.SemaphoreType.DMA,
        pltpu.SemaphoreType.DMA,
    ],
)
def _sc_gather(o_hbm, p0_hbm, p1_hbm, g0_hbm, g1_hbm, i0_v, i1_v,
               b0a, b1a, b0b, b1b, sga, sgb, ssa, ssb):
    wid = lax.axis_index("s") * 2 + lax.axis_index("c")
    pltpu.sync_copy(p0_hbm.at[wid], i0_v)
    pltpu.sync_copy(p1_hbm.at[wid], i1_v)
    sets = ((b0a, b1a, sga), (b0b, b1b, sgb))
    stsems = (ssa, ssb)
    base0 = wid * TOK_PER_W
    gaths = [None] * NCH
    stores = [None] * NCH

    def _issue(ch):
        b0, b1, sg = sets[ch % 2]
        return (pltpu.async_copy(o_hbm.at[i0_v.at[ch]], b0, sg),
                pltpu.async_copy(o_hbm.at[i1_v.at[ch]], b1, sg))

    gaths[0] = _issue(0)
    for ch in range(NCH):
        b0, b1, _ = sets[ch % 2]
        gaths[ch][0].wait()
        gaths[ch][1].wait()
        if ch + 1 < NCH:
            if ch >= 1:
                stores[ch - 1][0].wait()
                stores[ch - 1][1].wait()
            gaths[ch + 1] = _issue(ch + 1)
        base = base0 + ch * CH
        stores[ch] = (
            pltpu.async_copy(b0, g0_hbm.at[pl.ds(base, CH)], stsems[ch % 2]),
            pltpu.async_copy(b1, g1_hbm.at[pl.ds(base, CH)], stsems[ch % 2]),
        )
    for ch in (NCH - 2, NCH - 1):
        stores[ch][0].wait()
        stores[ch][1].wait()


_SQRT1_2 = float(1.0 / np.sqrt(2.0))


def _gelu_exact(x):
    return 0.5 * x * (1.0 + lax.erf(x * _SQRT1_2))


def _mlp_body(eids_ref, rb_ref, vld_ref, x_ref, w1_ref, b1_ref, w2_ref, b2_ref,
              o_ref, w1bf, w2bf, preve):
    i = pl.program_id(0)
    e = eids_ref[i]

    @pl.when((i == 0) | (e != preve[0]))
    def _cast():
        w1bf[...] = w1_ref[0].astype(jnp.bfloat16)
        w2bf[...] = w2_ref[0].astype(jnp.bfloat16)
        preve[0] = e

    @pl.when(vld_ref[i] == 1)
    def _compute():
        xb = x_ref[...].astype(jnp.bfloat16)
        h = jnp.dot(xb, w1bf[...], preferred_element_type=jnp.float32)
        h = _gelu_exact(h + b1_ref[0])
        o = jnp.dot(h.astype(jnp.bfloat16), w2bf[...],
                    preferred_element_type=jnp.float32)
        o_ref[...] = o + b2_ref[0]


def _grouped_mlp(x_sorted, W1, b1r, W2, b2r, eids, rb, vld):
    grid_spec = pltpu.PrefetchScalarGridSpec(
        num_scalar_prefetch=3,
        grid=(NT,),
        in_specs=[
            pl.BlockSpec((TM, D_IN), lambda i, eids, rb, vld: (rb[i], 0)),
            pl.BlockSpec((1, D_IN, D_HID), lambda i, eids, rb, vld: (eids[i], 0, 0)),
            pl.BlockSpec((1, 1, D_HID), lambda i, eids, rb, vld: (eids[i], 0, 0)),
            pl.BlockSpec((1, D_HID, D_OUT), lambda i, eids, rb, vld: (eids[i], 0, 0)),
            pl.BlockSpec((1, 1, D_OUT), lambda i, eids, rb, vld: (eids[i], 0, 0)),
        ],
        out_specs=pl.BlockSpec((TM, D_OUT), lambda i, eids, rb, vld: (rb[i], 0)),
        scratch_shapes=[
            pltpu.VMEM((D_IN, D_HID), jnp.bfloat16),
            pltpu.VMEM((D_HID, D_OUT), jnp.bfloat16),
            pltpu.SMEM((1,), jnp.int32),
        ],
    )
    return pl.pallas_call(
        _mlp_body,
        grid_spec=grid_spec,
        out_shape=jax.ShapeDtypeStruct((NPAD, D_OUT), jnp.float32),
    )(eids, rb, vld, x_sorted, W1, b1r, W2, b2r)


def _combine_body(g0_ref, g1_ref, w_ref, y_ref):
    w = w_ref[...]
    y_ref[...] = g0_ref[...] * w[:, 0:1] + g1_ref[...] * w[:, 1:2]


def _combine(g0, g1, w2d):
    grid = T // TM
    return pl.pallas_call(
        _combine_body,
        grid=(grid,),
        in_specs=[
            pl.BlockSpec((TM, D_OUT), lambda i: (i, 0)),
            pl.BlockSpec((TM, D_OUT), lambda i: (i, 0)),
            pl.BlockSpec((TM, TOP_K), lambda i: (i, 0)),
        ],
        out_specs=pl.BlockSpec((TM, D_OUT), lambda i: (i, 0)),
        out_shape=jax.ShapeDtypeStruct((T, D_OUT), jnp.float32),
    )(g0, g1, w2d)


def kernel(x, expert_mus, expert_log_sigmas, W1, b1, W2, b2):
    Bm, Tm, Din = x.shape
    xf = x.reshape(Tm, Din)

    lp, w2d, idx2d, pos2d, eids2, rb2, vld2 = _router(
        xf, expert_mus, expert_log_sigmas)
    eids, rb, vld = eids2.reshape(NT), rb2.reshape(NT), vld2.reshape(NT)

    p0 = pos2d[:, 0].reshape(NW, NCH, CH)
    p1 = pos2d[:, 1].reshape(NW, NCH, CH)

    x_sorted = _sc_scatter(xf, p0, p1)
    o_sorted = _grouped_mlp(x_sorted, W1, b1.reshape(E, 1, D_HID), W2,
                            b2.reshape(E, 1, D_OUT), eids, rb, vld)
    g0, g1 = _sc_gather(o_sorted, p0, p1)
    y = _combine(g0, g1, w2d)

    return (y.reshape(Bm, Tm, D_OUT),
            lp.reshape(Bm, Tm, E),
            w2d.reshape(Bm, Tm, TOP_K),
            idx2d.reshape(Bm, Tm, TOP_K))


# speedup vs baseline: 3.7564x; 1.0597x over previous
import functools

import jax
import jax.numpy as jnp
import numpy as np
from jax import lax
from jax.experimental import pallas as pl
from jax.experimental.pallas import tpu as pltpu
from jax.experimental.pallas import tpu_sc as plsc

E = 8
TOP_K = 2
D_IN = 1024
D_HID = 2048
D_OUT = 1024
T = 2048

TM = 256
NPAD = 6144
NT = NPAD // TM
NW = 32
TOK_PER_W = T // NW
CH = 16
NCH = TOK_PER_W // CH

_LOG2PI = 1.8378770664093453


def _router_body(x_ref, mu_ref, ls_ref, lp_ref, w_ref, idx_ref, pos_ref,
                 eids_ref, rb_ref, vld_ref):
    x = x_ref[...]
    cols = []
    for e in range(E):
        mu = mu_ref[e:e + 1, :]
        ls = ls_ref[e:e + 1, :]
        inv_sigma = jnp.exp(-ls)
        z = (x - mu) * inv_sigma
        s = jnp.sum(z * z, axis=1, keepdims=True)
        sls = jnp.sum(ls, axis=1, keepdims=True)
        cols.append(-0.5 * s - sls - (0.5 * _LOG2PI * D_IN))
    lp = jnp.concatenate(cols, axis=1)
    lp_ref[...] = lp

    iota_e = lax.broadcasted_iota(jnp.int32, (T, E), 1)
    m1 = jnp.max(lp, axis=1, keepdims=True)
    i1 = jnp.min(jnp.where(lp == m1, iota_e, E), axis=1, keepdims=True)
    masked = jnp.where(iota_e == i1, -jnp.inf, lp)
    m2 = jnp.max(masked, axis=1, keepdims=True)
    i2 = jnp.min(jnp.where(masked == m2, iota_e, E), axis=1, keepdims=True)

    p2 = jnp.exp(m2 - m1)
    inv = 1.0 / (1.0 + p2)
    w_ref[...] = jnp.concatenate([inv, p2 * inv], axis=1)
    idx_ref[...] = jnp.concatenate([i1, i2], axis=1)

    c1 = (iota_e == i1).astype(jnp.int32)
    c2 = (iota_e == i2).astype(jnp.int32)
    a = c1 + c2
    s_inc = a
    k = 1
    while k < T:
        shifted = jnp.concatenate(
            [jnp.zeros((k, E), jnp.int32), s_inc[: T - k, :]], axis=0)
        s_inc = s_inc + shifted
        k *= 2
    s_exc = s_inc - a
    counts = s_inc[T - 1: T, :]

    padded = ((counts + (TM - 1)) // TM) * TM
    p_exc = jnp.concatenate(
        [jnp.zeros((1, 1), jnp.int32), padded[:, : E - 1]], axis=1)
    k = 1
    while k < E:
        shifted = jnp.concatenate(
            [jnp.zeros((1, k), jnp.int32), p_exc[:, : E - k]], axis=1)
        p_exc = p_exc + shifted
        k *= 2
    slot = s_exc + p_exc
    pos1 = jnp.sum(c1 * slot, axis=1, keepdims=True)
    pos2 = jnp.sum(c2 * slot, axis=1, keepdims=True)
    pos_ref[...] = jnp.concatenate([pos1, pos2], axis=1)

    ends = p_exc + padded
    rows = lax.broadcasted_iota(jnp.int32, (1, NT), 1) * TM
    eidv = jnp.zeros((1, NT), jnp.int32)
    e_last = jnp.zeros((1, 1), jnp.int32)
    total = ends[:, E - 1: E]
    for e in range(E):
        eidv = eidv + (rows >= ends[:, e: e + 1]).astype(jnp.int32)
        e_last = e_last + (total - TM >= ends[:, e: e + 1]).astype(jnp.int32)
    valid = (rows < total).astype(jnp.int32)
    iota_t = lax.broadcasted_iota(jnp.int32, (1, NT), 1)
    rb_ref[...] = jnp.where(valid == 1, iota_t, total // TM - 1)
    eids_ref[...] = jnp.where(valid == 1, eidv, e_last)
    vld_ref[...] = valid


def _router(xf, mus, lsig):
    return pl.pallas_call(
        _router_body,
        out_shape=(
            jax.ShapeDtypeStruct((T, E), jnp.float32),
            jax.ShapeDtypeStruct((T, TOP_K), jnp.float32),
            jax.ShapeDtypeStruct((T, TOP_K), jnp.int32),
            jax.ShapeDtypeStruct((T, TOP_K), jnp.int32),
            jax.ShapeDtypeStruct((1, NT), jnp.int32),
            jax.ShapeDtypeStruct((1, NT), jnp.int32),
            jax.ShapeDtypeStruct((1, NT), jnp.int32),
        ),
    )(xf, mus, lsig)


_SC_MESH = plsc.VectorSubcoreMesh(core_axis_name="c", subcore_axis_name="s")


@functools.partial(
    pl.kernel,
    mesh=_SC_MESH,
    out_type=jax.ShapeDtypeStruct((NPAD, D_IN), jnp.float32),
    scratch_types=[
        pltpu.VMEM((NCH, CH), jnp.int32),
        pltpu.VMEM((NCH, CH), jnp.int32),
        pltpu.VMEM((CH, D_IN), jnp.float32),
        pltpu.VMEM((CH, D_IN), jnp.float32),
        pltpu.SemaphoreType.DMA,
        pltpu.SemaphoreType.DMA,
        pltpu.SemaphoreType.DMA,
        pltpu.SemaphoreType.DMA,
    ],
)
def _sc_scatter(x_hbm, p0_hbm, p1_hbm, out_hbm, i0_v, i1_v, xb0, xb1,
                sl0, sl1, ss0, ss1):
    wid = lax.axis_index("s") * 2 + lax.axis_index("c")
    pltpu.sync_copy(p0_hbm.at[wid], i0_v)
    pltpu.sync_copy(p1_hbm.at[wid], i1_v)
    bufs = (xb0, xb1)
    lsems = (sl0, sl1)
    ssems = (ss0, ss1)
    loads = [None] * NCH
    scats = [None] * NCH
    base0 = wid * TOK_PER_W
    loads[0] = pltpu.async_copy(x_hbm.at[pl.ds(base0, CH)], xb0, sl0)
    for ch in range(NCH):
        b = bufs[ch % 2]
        loads[ch].wait()
        if ch + 1 < NCH:
            if ch >= 1:
                scats[ch - 1][0].wait()
                scats[ch - 1][1].wait()
            loads[ch + 1] = pltpu.async_copy(
                x_hbm.at[pl.ds(base0 + (ch + 1) * CH, CH)],
                bufs[(ch + 1) % 2], lsems[(ch + 1) % 2])
        scats[ch] = (
            pltpu.async_copy(b, out_hbm.at[i0_v.at[ch]], ssems[ch % 2]),
            pltpu.async_copy(b, out_hbm.at[i1_v.at[ch]], ssems[ch % 2]),
        )
    for ch in (NCH - 2, NCH - 1):
        scats[ch][0].wait()
        scats[ch][1].wait()


@functools.partial(
    pl.kernel,
    mesh=_SC_MESH,
    out_type=(
        jax.ShapeDtypeStruct((T, D_OUT), jnp.float32),
        jax.ShapeDtypeStruct((T, D_OUT), jnp.float32),
    ),
    scratch_types=[
        pltpu.VMEM((NCH, CH), jnp.int32),
        pltpu.VMEM((NCH, CH), jnp.int32),
        pltpu.VMEM((CH, D_OUT), jnp.float32),
        pltpu.VMEM((CH, D_OUT), jnp.float32),
        pltpu.VMEM((CH, D_OUT), jnp.float32),
        pltpu.VMEM((CH, D_OUT), jnp.float32),
        pltpu.SemaphoreType.DMA,
        pltpu.SemaphoreType.DMA,
        pltpu.SemaphoreType.DMA,
        pltpu.SemaphoreType.DMA,
    ],
)
def _sc_gather(o_hbm, p0_hbm, p1_hbm, g0_hbm, g1_hbm, i0_v, i1_v,
               b0a, b1a, b0b, b1b, sga, sgb, ssa, ssb):
    wid = lax.axis_index("s") * 2 + lax.axis_index("c")
    pltpu.sync_copy(p0_hbm.at[wid], i0_v)
    pltpu.sync_copy(p1_hbm.at[wid], i1_v)
    sets = ((b0a, b1a, sga), (b0b, b1b, sgb))
    stsems = (ssa, ssb)
    base0 = wid * TOK_PER_W
    gaths = [None] * NCH
    stores = [None] * NCH

    def _issue(ch):
        b0, b1, sg = sets[ch % 2]
        return (pltpu.async_copy(o_hbm.at[i0_v.at[ch]], b0, sg),
                pltpu.async_copy(o_hbm.at[i1_v.at[ch]], b1, sg))

    gaths[0] = _issue(0)
    for ch in range(NCH):
        b0, b1, _ = sets[ch % 2]
        gaths[ch][0].wait()
        gaths[ch][1].wait()
        if ch + 1 < NCH:
            if ch >= 1:
                stores[ch - 1][0].wait()
                stores[ch - 1][1].wait()
            gaths[ch + 1] = _issue(ch + 1)
        base = base0 + ch * CH
        stores[ch] = (
            pltpu.async_copy(b0, g0_hbm.at[pl.ds(base, CH)], stsems[ch % 2]),
            pltpu.async_copy(b1, g1_hbm.at[pl.ds(base, CH)], stsems[ch % 2]),
        )
    for ch in (NCH - 2, NCH - 1):
        stores[ch][0].wait()
        stores[ch][1].wait()


_SQRT1_2 = float(1.0 / np.sqrt(2.0))


def _gelu_exact(x):
    return 0.5 * x * (1.0 + lax.erf(x * _SQRT1_2))


def _mlp_body(eids_ref, rb_ref, vld_ref, x_ref, w1_ref, b1_ref, w2_ref, b2_ref,
              o_ref, w1bf, w2bf, preve):
    i = pl.program_id(0)
    e = eids_ref[i]

    @pl.when((i == 0) | (e != preve[0]))
    def _cast():
        w1bf[...] = w1_ref[0].astype(jnp.bfloat16)
        w2bf[...] = w2_ref[0].astype(jnp.bfloat16)
        preve[0] = e

    @pl.when(vld_ref[i] == 1)
    def _compute():
        xb = x_ref[...].astype(jnp.bfloat16)
        h = jnp.dot(xb, w1bf[...], preferred_element_type=jnp.float32)
        h = _gelu_exact(h + b1_ref[0])
        o = jnp.dot(h.astype(jnp.bfloat16), w2bf[...],
                    preferred_element_type=jnp.float32)
        o_ref[...] = o + b2_ref[0]


def _grouped_mlp(x_sorted, W1, b1r, W2, b2r, eids, rb, vld):
    grid_spec = pltpu.PrefetchScalarGridSpec(
        num_scalar_prefetch=3,
        grid=(NT,),
        in_specs=[
            pl.BlockSpec((TM, D_IN), lambda i, eids, rb, vld: (rb[i], 0)),
            pl.BlockSpec((1, D_IN, D_HID), lambda i, eids, rb, vld: (eids[i], 0, 0)),
            pl.BlockSpec((1, 1, D_HID), lambda i, eids, rb, vld: (eids[i], 0, 0)),
            pl.BlockSpec((1, D_HID, D_OUT), lambda i, eids, rb, vld: (eids[i], 0, 0)),
            pl.BlockSpec((1, 1, D_OUT), lambda i, eids, rb, vld: (eids[i], 0, 0)),
        ],
        out_specs=pl.BlockSpec((TM, D_OUT), lambda i, eids, rb, vld: (rb[i], 0)),
        scratch_shapes=[
            pltpu.VMEM((D_IN, D_HID), jnp.bfloat16),
            pltpu.VMEM((D_HID, D_OUT), jnp.bfloat16),
            pltpu.SMEM((1,), jnp.int32),
        ],
    )
    return pl.pallas_call(
        _mlp_body,
        grid_spec=grid_spec,
        out_shape=jax.ShapeDtypeStruct((NPAD, D_OUT), jnp.float32),
    )(eids, rb, vld, x_sorted, W1, b1r, W2, b2r)


def _combine_body(g0_ref, g1_ref, w_ref, y_ref):
    w = w_ref[...]
    y_ref[...] = g0_ref[...] * w[:, 0:1] + g1_ref[...] * w[:, 1:2]


def _combine(g0, g1, w2d):
    grid = T // TM
    return pl.pallas_call(
        _combine_body,
        grid=(grid,),
        in_specs=[
            pl.BlockSpec((TM, D_OUT), lambda i: (i, 0)),
            pl.BlockSpec((TM, D_OUT), lambda i: (i, 0)),
            pl.BlockSpec((TM, TOP_K), lambda i: (i, 0)),
        ],
        out_specs=pl.BlockSpec((TM, D_OUT), lambda i: (i, 0)),
        out_shape=jax.ShapeDtypeStruct((T, D_OUT), jnp.float32),
    )(g0, g1, w2d)


def kernel(x, expert_mus, expert_log_sigmas, W1, b1, W2, b2):
    Bm, Tm, Din = x.shape
    xf = x.reshape(Tm, Din)

    lp, w2d, idx2d, pos2d, eids2, rb2, vld2 = _router(
        xf, expert_mus, expert_log_sigmas)
    eids, rb, vld = eids2.reshape(NT), rb2.reshape(NT), vld2.reshape(NT)

    p0 = pos2d[:, 0].reshape(NW, NCH, CH)
    p1 = pos2d[:, 1].reshape(NW, NCH, CH)

    x_sorted = _sc_scatter(xf, p0, p1)
    o_sorted = _grouped_mlp(x_sorted, W1, b1.reshape(E, 1, D_HID), W2,
                            b2.reshape(E, 1, D_OUT), eids, rb, vld)
    g0, g1 = _sc_gather(o_sorted, p0, p1)
    y = _combine(g0, g1, w2d)

    return (y.reshape(Bm, Tm, D_OUT),
            lp.reshape(Bm, Tm, E),
            w2d.reshape(Bm, Tm, TOP_K),
            idx2d.reshape(Bm, Tm, TOP_K))


# speedup vs baseline: 3.9882x; 1.0617x over previous
import functools

import jax
import jax.numpy as jnp
import numpy as np
from jax import lax
from jax.experimental import pallas as pl
from jax.experimental.pallas import tpu as pltpu
from jax.experimental.pallas import tpu_sc as plsc

E = 8
TOP_K = 2
D_IN = 1024
D_HID = 2048
D_OUT = 1024
T = 2048

TM = 256
NPAD = 6144
NT = NPAD // TM
NW = 32
TOK_PER_W = T // NW
CH = 16
NCH = TOK_PER_W // CH

_LOG2PI = 1.8378770664093453


def _router_body(x_ref, mu_ref, ls_ref, lp_ref, w_ref, idx_ref, pos_ref,
                 eids_ref, rb_ref, vld_ref, fof_ref, nxte_ref, hvn_ref):
    x = x_ref[...]
    cols = []
    for e in range(E):
        mu = mu_ref[e:e + 1, :]
        ls = ls_ref[e:e + 1, :]
        inv_sigma = jnp.exp(-ls)
        z = (x - mu) * inv_sigma
        s = jnp.sum(z * z, axis=1, keepdims=True)
        sls = jnp.sum(ls, axis=1, keepdims=True)
        cols.append(-0.5 * s - sls - (0.5 * _LOG2PI * D_IN))
    lp = jnp.concatenate(cols, axis=1)
    lp_ref[...] = lp

    iota_e = lax.broadcasted_iota(jnp.int32, (T, E), 1)
    m1 = jnp.max(lp, axis=1, keepdims=True)
    i1 = jnp.min(jnp.where(lp == m1, iota_e, E), axis=1, keepdims=True)
    masked = jnp.where(iota_e == i1, -jnp.inf, lp)
    m2 = jnp.max(masked, axis=1, keepdims=True)
    i2 = jnp.min(jnp.where(masked == m2, iota_e, E), axis=1, keepdims=True)

    p2 = jnp.exp(m2 - m1)
    inv = 1.0 / (1.0 + p2)
    w_ref[...] = jnp.concatenate([inv, p2 * inv], axis=1)
    idx_ref[...] = jnp.concatenate([i1, i2], axis=1)

    c1 = (iota_e == i1).astype(jnp.int32)
    c2 = (iota_e == i2).astype(jnp.int32)
    a = c1 + c2
    s_inc = a
    k = 1
    while k < T:
        shifted = jnp.concatenate(
            [jnp.zeros((k, E), jnp.int32), s_inc[: T - k, :]], axis=0)
        s_inc = s_inc + shifted
        k *= 2
    s_exc = s_inc - a
    counts = s_inc[T - 1: T, :]

    padded = ((counts + (TM - 1)) // TM) * TM
    p_exc = jnp.concatenate(
        [jnp.zeros((1, 1), jnp.int32), padded[:, : E - 1]], axis=1)
    k = 1
    while k < E:
        shifted = jnp.concatenate(
            [jnp.zeros((1, k), jnp.int32), p_exc[:, : E - k]], axis=1)
        p_exc = p_exc + shifted
        k *= 2
    slot = s_exc + p_exc
    pos1 = jnp.sum(c1 * slot, axis=1, keepdims=True)
    pos2 = jnp.sum(c2 * slot, axis=1, keepdims=True)
    pos_ref[...] = jnp.concatenate([pos1, pos2], axis=1)

    ends = p_exc + padded
    rows = lax.broadcasted_iota(jnp.int32, (1, NT), 1) * TM
    eidv = jnp.zeros((1, NT), jnp.int32)
    e_last = jnp.zeros((1, 1), jnp.int32)
    total = ends[:, E - 1: E]
    for e in range(E):
        eidv = eidv + (rows >= ends[:, e: e + 1]).astype(jnp.int32)
        e_last = e_last + (total - TM >= ends[:, e: e + 1]).astype(jnp.int32)
    valid = (rows < total).astype(jnp.int32)
    iota_t = lax.broadcasted_iota(jnp.int32, (1, NT), 1)
    rb_ref[...] = jnp.where(valid == 1, iota_t, total // TM - 1)
    eids_ref[...] = jnp.where(valid == 1, eidv, e_last)
    vld_ref[...] = valid

    prev_e = jnp.concatenate(
        [jnp.full((1, 1), -1, jnp.int32), eidv[:, : NT - 1]], axis=1)
    fof_ref[...] = ((eidv != prev_e) & (valid == 1)).astype(jnp.int32)
    end_sel = jnp.zeros((1, NT), jnp.int32)
    for e in range(E):
        end_sel = end_sel + jnp.where(eidv == e, ends[:, e: e + 1], 0)
    nxte = jnp.zeros((1, NT), jnp.int32)
    for e in range(E):
        nxte = nxte + (end_sel >= ends[:, e: e + 1]).astype(jnp.int32)
    nxte_ref[...] = jnp.minimum(nxte, E - 1)
    hvn_ref[...] = ((end_sel < total) & (valid == 1)).astype(jnp.int32)


def _router(xf, mus, lsig):
    return pl.pallas_call(
        _router_body,
        out_shape=(
            jax.ShapeDtypeStruct((T, E), jnp.float32),
            jax.ShapeDtypeStruct((T, TOP_K), jnp.float32),
            jax.ShapeDtypeStruct((T, TOP_K), jnp.int32),
            jax.ShapeDtypeStruct((T, TOP_K), jnp.int32),
            jax.ShapeDtypeStruct((1, NT), jnp.int32),
            jax.ShapeDtypeStruct((1, NT), jnp.int32),
            jax.ShapeDtypeStruct((1, NT), jnp.int32),
            jax.ShapeDtypeStruct((1, NT), jnp.int32),
            jax.ShapeDtypeStruct((1, NT), jnp.int32),
            jax.ShapeDtypeStruct((1, NT), jnp.int32),
        ),
    )(xf, mus, lsig)


_SC_MESH = plsc.VectorSubcoreMesh(core_axis_name="c", subcore_axis_name="s")


@functools.partial(
    pl.kernel,
    mesh=_SC_MESH,
    out_type=jax.ShapeDtypeStruct((NPAD, D_IN), jnp.float32),
    scratch_types=[
        pltpu.VMEM((NCH, CH), jnp.int32),
        pltpu.VMEM((NCH, CH), jnp.int32),
        pltpu.VMEM((CH, D_IN), jnp.float32),
        pltpu.VMEM((CH, D_IN), jnp.float32),
        pltpu.SemaphoreType.DMA,
        pltpu.SemaphoreType.DMA,
        pltpu.SemaphoreType.DMA,
        pltpu.SemaphoreType.DMA,
    ],
)
def _sc_scatter(x_hbm, p0_hbm, p1_hbm, out_hbm, i0_v, i1_v, xb0, xb1,
                sl0, sl1, ss0, ss1):
    wid = lax.axis_index("s") * 2 + lax.axis_index("c")
    pltpu.sync_copy(p0_hbm.at[wid], i0_v)
    pltpu.sync_copy(p1_hbm.at[wid], i1_v)
    bufs = (xb0, xb1)
    lsems = (sl0, sl1)
    ssems = (ss0, ss1)
    loads = [None] * NCH
    scats = [None] * NCH
    base0 = wid * TOK_PER_W
    loads[0] = pltpu.async_copy(x_hbm.at[pl.ds(base0, CH)], xb0, sl0)
    for ch in range(NCH):
        b = bufs[ch % 2]
        loads[ch].wait()
        if ch + 1 < NCH:
            if ch >= 1:
                scats[ch - 1][0].wait()
                scats[ch - 1][1].wait()
            loads[ch + 1] = pltpu.async_copy(
                x_hbm.at[pl.ds(base0 + (ch + 1) * CH, CH)],
                bufs[(ch + 1) % 2], lsems[(ch + 1) % 2])
        scats[ch] = (
            pltpu.async_copy(b, out_hbm.at[i0_v.at[ch]], ssems[ch % 2]),
            pltpu.async_copy(b, out_hbm.at[i1_v.at[ch]], ssems[ch % 2]),
        )
    for ch in (NCH - 2, NCH - 1):
        scats[ch][0].wait()
        scats[ch][1].wait()


@functools.partial(
    pl.kernel,
    mesh=_SC_MESH,
    out_type=(
        jax.ShapeDtypeStruct((T, D_OUT), jnp.float32),
        jax.ShapeDtypeStruct((T, D_OUT), jnp.float32),
    ),
    scratch_types=[
        pltpu.VMEM((NCH, CH), jnp.int32),
        pltpu.VMEM((NCH, CH), jnp.int32),
        pltpu.VMEM((CH, D_OUT), jnp.float32),
        pltpu.VMEM((CH, D_OUT), jnp.float32),
        pltpu.VMEM((CH, D_OUT), jnp.float32),
        pltpu.VMEM((CH, D_OUT), jnp.float32),
        pltpu.SemaphoreType.DMA,
        pltpu.SemaphoreType.DMA,
        pltpu.SemaphoreType.DMA,
        pltpu.SemaphoreType.DMA,
    ],
)
def _sc_gather(o_hbm, p0_hbm, p1_hbm, g0_hbm, g1_hbm, i0_v, i1_v,
               b0a, b1a, b0b, b1b, sga, sgb, ssa, ssb):
    wid = lax.axis_index("s") * 2 + lax.axis_index("c")
    pltpu.sync_copy(p0_hbm.at[wid], i0_v)
    pltpu.sync_copy(p1_hbm.at[wid], i1_v)
    sets = ((b0a, b1a, sga), (b0b, b1b, sgb))
    stsems = (ssa, ssb)
    base0 = wid * TOK_PER_W
    gaths = [None] * NCH
    stores = [None] * NCH

    def _issue(ch):
        b0, b1, sg = sets[ch % 2]
        return (pltpu.async_copy(o_hbm.at[i0_v.at[ch]], b0, sg),
                pltpu.async_copy(o_hbm.at[i1_v.at[ch]], b1, sg))

    gaths[0] = _issue(0)
    for ch in range(NCH):
        b0, b1, _ = sets[ch % 2]
        gaths[ch][0].wait()
        gaths[ch][1].wait()
        if ch + 1 < NCH:
            if ch >= 1:
                stores[ch - 1][0].wait()
                stores[ch - 1][1].wait()
            gaths[ch + 1] = _issue(ch + 1)
        base = base0 + ch * CH
        stores[ch] = (
            pltpu.async_copy(b0, g0_hbm.at[pl.ds(base, CH)], stsems[ch % 2]),
            pltpu.async_copy(b1, g1_hbm.at[pl.ds(base, CH)], stsems[ch % 2]),
        )
    for ch in (NCH - 2, NCH - 1):
        stores[ch][0].wait()
        stores[ch][1].wait()


_SQRT1_2 = float(1.0 / np.sqrt(2.0))


def _gelu_exact(x):
    return 0.5 * x * (1.0 + lax.erf(x * _SQRT1_2))


def _mlp_body(eids_ref, rb_ref, vld_ref, fof_ref, nxte_ref, hvn_ref,
              x_ref, w1_ref, b1_ref, w2_ref, b2_ref,
              o_ref, w1r, w2r, w1bf, w2bf, slot_ref, sem1, sem2):
    i = pl.program_id(0)
    e = eids_ref[i]

    def _start(dst_e, dst_slot):
        pltpu.make_async_copy(w1_ref.at[dst_e], w1r.at[dst_slot],
                              sem1.at[dst_slot]).start()
        pltpu.make_async_copy(w2_ref.at[dst_e], w2r.at[dst_slot],
                              sem2.at[dst_slot]).start()

    def _wait(dst_e, dst_slot):
        pltpu.make_async_copy(w1_ref.at[dst_e], w1r.at[dst_slot],
                              sem1.at[dst_slot]).wait()
        pltpu.make_async_copy(w2_ref.at[dst_e], w2r.at[dst_slot],
                              sem2.at[dst_slot]).wait()

    @pl.when((fof_ref[i] == 1) & (i == 0))
    def _first_segment():
        _start(e, 0)
        _wait(e, 0)
        slot_ref[0] = 0

    @pl.when((fof_ref[i] == 1) & (i > 0))
    def _next_segment():
        ns = 1 - slot_ref[0]
        _wait(e, ns)
        slot_ref[0] = ns

    @pl.when(fof_ref[i] == 1)
    def _cast_and_prefetch():
        s = slot_ref[0]
        w1bf[...] = w1r[s].astype(jnp.bfloat16)
        w2bf[...] = w2r[s].astype(jnp.bfloat16)

        @pl.when(hvn_ref[i] == 1)
        def _prefetch_next():
            _start(nxte_ref[i], 1 - s)

    @pl.when(vld_ref[i] == 1)
    def _compute():
        xb = x_ref[...].astype(jnp.bfloat16)
        h = jnp.dot(xb, w1bf[...], preferred_element_type=jnp.float32)
        h = _gelu_exact(h + b1_ref[0])
        o = jnp.dot(h.astype(jnp.bfloat16), w2bf[...],
                    preferred_element_type=jnp.float32)
        o_ref[...] = o + b2_ref[0]


def _grouped_mlp(x_sorted, W1, b1r, W2, b2r, eids, rb, vld, fof, nxte, hvn):
    grid_spec = pltpu.PrefetchScalarGridSpec(
        num_scalar_prefetch=6,
        grid=(NT,),
        in_specs=[
            pl.BlockSpec((TM, D_IN), lambda i, *s: (s[1][i], 0)),
            pl.BlockSpec(memory_space=pltpu.MemorySpace.HBM),
            pl.BlockSpec((1, 1, D_HID), lambda i, *s: (s[0][i], 0, 0)),
            pl.BlockSpec(memory_space=pltpu.MemorySpace.HBM),
            pl.BlockSpec((1, 1, D_OUT), lambda i, *s: (s[0][i], 0, 0)),
        ],
        out_specs=pl.BlockSpec((TM, D_OUT), lambda i, *s: (s[1][i], 0)),
        scratch_shapes=[
            pltpu.VMEM((2, D_IN, D_HID), jnp.float32),
            pltpu.VMEM((2, D_HID, D_OUT), jnp.float32),
            pltpu.VMEM((D_IN, D_HID), jnp.bfloat16),
            pltpu.VMEM((D_HID, D_OUT), jnp.bfloat16),
            pltpu.SMEM((1,), jnp.int32),
            pltpu.SemaphoreType.DMA((2,)),
            pltpu.SemaphoreType.DMA((2,)),
        ],
    )
    return pl.pallas_call(
        _mlp_body,
        grid_spec=grid_spec,
        out_shape=jax.ShapeDtypeStruct((NPAD, D_OUT), jnp.float32),
    )(eids, rb, vld, fof, nxte, hvn, x_sorted, W1, b1r, W2, b2r)


def _combine_body(g0_ref, g1_ref, w_ref, y_ref):
    w = w_ref[...]
    y_ref[...] = g0_ref[...] * w[:, 0:1] + g1_ref[...] * w[:, 1:2]


def _combine(g0, g1, w2d):
    grid = T // TM
    return pl.pallas_call(
        _combine_body,
        grid=(grid,),
        in_specs=[
            pl.BlockSpec((TM, D_OUT), lambda i: (i, 0)),
            pl.BlockSpec((TM, D_OUT), lambda i: (i, 0)),
            pl.BlockSpec((TM, TOP_K), lambda i: (i, 0)),
        ],
        out_specs=pl.BlockSpec((TM, D_OUT), lambda i: (i, 0)),
        out_shape=jax.ShapeDtypeStruct((T, D_OUT), jnp.float32),
    )(g0, g1, w2d)


def kernel(x, expert_mus, expert_log_sigmas, W1, b1, W2, b2):
    Bm, Tm, Din = x.shape
    xf = x.reshape(Tm, Din)

    lp, w2d, idx2d, pos2d, eids2, rb2, vld2, fof2, nxte2, hvn2 = _router(
        xf, expert_mus, expert_log_sigmas)
    eids, rb, vld = eids2.reshape(NT), rb2.reshape(NT), vld2.reshape(NT)
    fof, nxte, hvn = fof2.reshape(NT), nxte2.reshape(NT), hvn2.reshape(NT)

    p0 = pos2d[:, 0].reshape(NW, NCH, CH)
    p1 = pos2d[:, 1].reshape(NW, NCH, CH)

    x_sorted = _sc_scatter(xf, p0, p1)
    o_sorted = _grouped_mlp(x_sorted, W1, b1.reshape(E, 1, D_HID), W2,
                            b2.reshape(E, 1, D_OUT), eids, rb, vld, fof,
                            nxte, hvn)
    g0, g1 = _sc_gather(o_sorted, p0, p1)
    y = _combine(g0, g1, w2d)

    return (y.reshape(Bm, Tm, D_OUT),
            lp.reshape(Bm, Tm, E),
            w2d.reshape(Bm, Tm, TOP_K),
            idx2d.reshape(Bm, Tm, TOP_K))
